# baseline (speedup 1.0000x reference)
import jax
import jax.numpy as jnp
from jax import lax
from jax.experimental import pallas as pl
from jax.experimental.pallas import tpu as pltpu

COLUMNS = 2048
CELLS_PER_COLUMN = 8
NUM_CELLS = COLUMNS * CELLS_PER_COLUMN
SEGMENTS = 16
SYNAPSES = 32
SEGSYN = SEGMENTS * SYNAPSES
ACTIVATION_THRESHOLD = 5
CONNECTED_PERMANENCE = 0.8
VOLATILE_LR = 0.1
BATCH = 16
_BLK_CELLS = 1024
_BLK_COLS = _BLK_CELLS // CELLS_PER_COLUMN
_NBLK = NUM_CELLS // _BLK_CELLS
_ROWS = _BLK_CELLS * SEGSYN // 128
_RCHUNK = 32


def _expand_mat():
    c = lax.broadcasted_iota(jnp.int32, (_BLK_COLS, _BLK_CELLS), 0)
    n = lax.broadcasted_iota(jnp.int32, (_BLK_COLS, _BLK_CELLS), 1) // CELLS_PER_COLUMN
    return (c == n).astype(jnp.bfloat16)


def _reduce_mat():
    n = lax.broadcasted_iota(jnp.int32, (_BLK_CELLS, _BLK_COLS), 0) // CELLS_PER_COLUMN
    c = lax.broadcasted_iota(jnp.int32, (_BLK_CELLS, _BLK_COLS), 1)
    return (n == c).astype(jnp.bfloat16)


def _seg_mat():
    j = lax.broadcasted_iota(jnp.int32, (SEGSYN, SEGMENTS), 0) // SYNAPSES
    s = lax.broadcasted_iota(jnp.int32, (SEGSYN, SEGMENTS), 1)
    return (j == s).astype(jnp.bfloat16)


def _pack_rows(act):
    p = jnp.zeros((1, NUM_CELLS), jnp.float32)
    for b in range(BATCH):
        p = p + act[b:b + 1, :] * float(1 << b)
    return p.reshape(128, 128)


def _byte_tables(packed):
    pi = packed.astype(jnp.int32)
    lo = (pi & 255).astype(jnp.bfloat16)
    hi = ((pi >> 8) & 255).astype(jnp.bfloat16)
    return lo, hi


def _onehots(idx):
    lane = lax.broadcasted_iota(jnp.int32, (_RCHUNK, 128), 1)
    oh_hi = ((idx >> 7) == lane).astype(jnp.bfloat16)
    oh_lo = ((idx & 127) == lane).astype(jnp.float32)
    return oh_hi, oh_lo


def _gather_tbl(oh_hi, oh_lo, tbl):
    rows = jnp.dot(oh_hi, tbl, preferred_element_type=jnp.float32)
    return jnp.sum(rows * oh_lo, axis=1, keepdims=True)


def _bhot(b):
    lane = lax.broadcasted_iota(jnp.int32, (1, BATCH), 1)
    return (lane == b).astype(jnp.float32)


def _seg_to_cells(ovacc):
    predseg = (ovacc >= ACTIVATION_THRESHOLD).astype(jnp.bfloat16)
    l64 = lax.broadcasted_iota(jnp.int32, (64, BATCH), 0)
    bix = lax.broadcasted_iota(jnp.int32, (64, BATCH), 1)
    msel = ((l64 & 15) == bix).astype(jnp.bfloat16)
    scnt = jnp.dot(predseg, msel, preferred_element_type=jnp.float32)
    cix = lax.broadcasted_iota(jnp.int32, (_BLK_CELLS, _ROWS), 0)
    rix = lax.broadcasted_iota(jnp.int32, (_BLK_CELLS, _ROWS), 1) // 4
    m4 = (cix == rix).astype(jnp.bfloat16)
    cellcnt = jnp.dot(m4, scnt.astype(jnp.bfloat16),
                      preferred_element_type=jnp.float32)
    padded = jnp.concatenate(
        [cellcnt, jnp.zeros((_BLK_CELLS, 128 - BATCH), jnp.float32)], axis=1)
    t = jnp.transpose(padded)
    return (t[0:BATCH, :] > 0).astype(jnp.float32)


def _accumulate_overlaps(dist_s, mask_fn, tlo, thi, ovacc_ref):
    def body(t, _):
        r0 = t * _RCHUNK
        dchunk = dist_s[pl.ds(r0, _RCHUNK), :]
        accs = []
        for sg in range(4):
            acc16 = jnp.zeros((_RCHUNK, BATCH), jnp.float32)
            for j in range(sg * 32, (sg + 1) * 32):
                idx = dchunk[:, j:j + 1]
                oh_hi, oh_lo = _onehots(idx)
                glo = _gather_tbl(oh_hi, oh_lo, tlo)
                ghi = _gather_tbl(oh_hi, oh_lo, thi)
                g = (glo + 256.0 * ghi).astype(jnp.int32)
                m = g * mask_fn(r0, j, oh_hi, oh_lo)
                for b in range(BATCH):
                    acc16 = acc16 + ((m >> b) & 1).astype(jnp.float32) * _bhot(b)
            accs.append(acc16)
        ovacc_ref[pl.ds(r0, _RCHUNK), :] = jnp.concatenate(accs, axis=1)
        return 0

    lax.fori_loop(0, _ROWS // _RCHUNK, body, 0)


def _column_phase(pred_now, sdr, na_out_ref):
    mexp = _expand_mat()
    mred = _reduce_mat()
    colcnt = jnp.dot(pred_now.astype(jnp.bfloat16), mred,
                     preferred_element_type=jnp.float32)
    col_has = colcnt > 0
    colpred_exp = jnp.dot(col_has.astype(jnp.bfloat16), mexp,
                          preferred_element_type=jnp.float32)
    sdr_exp = jnp.dot(sdr.astype(jnp.bfloat16), mexp,
                      preferred_element_type=jnp.float32)
    na_out_ref[...] = sdr_exp * jnp.where(colpred_exp > 0, pred_now, 1.0)
    return jnp.sum(sdr, axis=1), jnp.sum(jnp.where(col_has, sdr, 0.0), axis=1)


def _phase_a(sdr_ref, mod_ref, prev_ref, vol_ref, cons_ref, dist_ref,
             na_ref, f2_ref, acc_ref,
             accs_ref, pred_s, dist_s, conn_s, ovacc_ref, sem):
    i = pl.program_id(0)
    delta = VOLATILE_LR * (jnp.sum(mod_ref[...]) / BATCH)
    vol = vol_ref[...]
    cons = cons_ref[...]
    eff = vol + cons
    kmat = _seg_mat()
    conn = eff >= CONNECTED_PERMANENCE
    cnt1 = jnp.dot(conn.astype(jnp.bfloat16), kmat,
                   preferred_element_type=jnp.float32)
    cand2 = (jnp.clip(vol + delta, 0.0, 1.0) + cons + 1e-5) >= CONNECTED_PERMANENCE
    cnt2 = jnp.dot(cand2.astype(jnp.bfloat16), kmat,
                   preferred_element_type=jnp.float32)
    n_flag1 = jnp.sum((cnt1 >= ACTIVATION_THRESHOLD).astype(jnp.float32))
    n_flag2 = jnp.sum((cnt2 >= ACTIVATION_THRESHOLD).astype(jnp.float32))
    f2_ref[...] = jnp.full((1, 1, BATCH), n_flag2, jnp.float32)

    pred_s[...] = jnp.zeros((BATCH, _BLK_CELLS), jnp.float32)

    @pl.when(n_flag1 > 0)
    def _():
        cp = pltpu.make_async_copy(
            dist_ref.at[pl.ds(i * _ROWS, _ROWS)], dist_s, sem)
        cp.start()
        cp.wait()
        tlo, thi = _byte_tables(_pack_rows(prev_ref[...]))
        conn_s[...] = conn.astype(jnp.int32).reshape(_ROWS, 128)

        def mask_fn(r0, j, oh_hi, oh_lo):
            return conn_s[pl.ds(r0, _RCHUNK), j:j + 1]

        _accumulate_overlaps(dist_s, mask_fn, tlo, thi, ovacc_ref)
        pred_s[...] = _seg_to_cells(ovacc_ref[...])

    pred_now = pred_s[...]
    na_part, np_part = _column_phase(pred_now, sdr_ref[...], na_ref)

    @pl.when(i == 0)
    def _():
        accs_ref[...] = jnp.zeros((8, BATCH), jnp.float32)

    accs_ref[0:1, :] += na_part.reshape(1, BATCH)
    accs_ref[1:2, :] += np_part.reshape(1, BATCH)

    @pl.when(i == _NBLK - 1)
    def _():
        nact = accs_ref[0:1, :]
        npred = accs_ref[1:2, :]
        acc_ref[...] = jnp.where(nact > 0,
                                 npred / jnp.maximum(nact, 1.0), 1.0)


def _phase_b(f2_ref, sdr_ref, mod_ref, na_full_ref, na_blk_ref, prev_ref,
             dist_ref, vol_ref, cons_ref,
             pred_ref,
             dist_s, vol_s, cons_s, w_s, ovacc_ref, sem):
    i = pl.program_id(0)
    pred_ref[...] = jnp.zeros((BATCH, _BLK_CELLS), jnp.float32)

    @pl.when(f2_ref[i] > 0)
    def _():
        base = i * _ROWS
        for src, dst in ((dist_ref, dist_s), (vol_ref, vol_s),
                         (cons_ref, cons_s)):
            cp = pltpu.make_async_copy(
                src.at[pl.ds(base, _ROWS)], dst, sem)
            cp.start()
            cp.wait()

        delta = VOLATILE_LR * (jnp.sum(mod_ref[...]) / BATCH)
        na_full = na_full_ref[...]
        tlo, thi = _byte_tables(_pack_rows(na_full))
        prevf = prev_ref[...]
        pany = jnp.zeros((1, NUM_CELLS), jnp.float32)
        for b in range(BATCH):
            pany = pany + prevf[b:b + 1, :]
        tpa = (pany > 0).astype(jnp.bfloat16).reshape(128, 128)

        na_blk = na_blk_ref[...]
        wcnt = lax.dot_general(
            na_blk.astype(jnp.bfloat16), jnp.ones((BATCH, 1), jnp.bfloat16),
            (((0,), (0,)), ((), ())), preferred_element_type=jnp.float32)
        wcol = (wcnt > 0).astype(jnp.bfloat16)
        rix = lax.broadcasted_iota(jnp.int32, (_ROWS, _BLK_CELLS), 0) // 4
        cix = lax.broadcasted_iota(jnp.int32, (_ROWS, _BLK_CELLS), 1)
        m4t = (rix == cix).astype(jnp.bfloat16)
        w4096 = jnp.dot(m4t, wcol, preferred_element_type=jnp.float32)
        w_s[...] = w4096 * jnp.ones((1, 128), jnp.float32)

        def mask_fn(r0, j, oh_hi, oh_lo):
            pp = _gather_tbl(oh_hi, oh_lo, tpa)
            vol_c = vol_s[pl.ds(r0, _RCHUNK), j:j + 1]
            cons_c = cons_s[pl.ds(r0, _RCHUNK), j:j + 1]
            w_c = w_s[pl.ds(r0, _RCHUNK), j:j + 1]
            nv = jnp.clip(
                vol_c + delta * w_c * pp - 0.1 * delta * w_c * (1.0 - pp),
                0.0, 1.0)
            return ((nv + cons_c) >= CONNECTED_PERMANENCE).astype(jnp.int32)

        _accumulate_overlaps(dist_s, mask_fn, tlo, thi, ovacc_ref)
        pred2 = _seg_to_cells(ovacc_ref[...])
        sdr_exp = jnp.dot(sdr_ref[...].astype(jnp.bfloat16), _expand_mat(),
                          preferred_element_type=jnp.float32)
        pred_ref[...] = pred2 * sdr_exp


def kernel(sdr_batch, modulation_signal_batch, prev_active_cells,
           distal_connections, volatile_permanences, consolidated_permanences):
    sdr_f = sdr_batch.astype(jnp.float32)
    mod2 = modulation_signal_batch.reshape(1, BATCH)
    prev_f = prev_active_cells.astype(jnp.float32)
    vol2 = volatile_permanences.reshape(NUM_CELLS, SEGSYN)
    cons2 = consolidated_permanences.reshape(NUM_CELLS, SEGSYN)
    dist_i32 = distal_connections.astype(jnp.int32)
    nrows = NUM_CELLS * SEGSYN // 128
    dist3 = dist_i32.reshape(nrows, 128)
    vol3 = volatile_permanences.reshape(nrows, 128)
    cons3 = consolidated_permanences.reshape(nrows, 128)

    new_active_f, f2, acc = pl.pallas_call(
        _phase_a,
        grid=(_NBLK,),
        in_specs=[
            pl.BlockSpec((BATCH, _BLK_COLS), lambda i: (0, i)),
            pl.BlockSpec((1, BATCH), lambda i: (0, 0)),
            pl.BlockSpec((BATCH, NUM_CELLS), lambda i: (0, 0)),
            pl.BlockSpec((_BLK_CELLS, SEGSYN), lambda i: (i, 0)),
            pl.BlockSpec((_BLK_CELLS, SEGSYN), lambda i: (i, 0)),
            pl.BlockSpec(memory_space=pl.ANY),
        ],
        out_specs=[
            pl.BlockSpec((BATCH, _BLK_CELLS), lambda i: (0, i)),
            pl.BlockSpec((1, 1, BATCH), lambda i: (i, 0, 0)),
            pl.BlockSpec((1, BATCH), lambda i: (0, 0)),
        ],
        out_shape=[
            jax.ShapeDtypeStruct((BATCH, NUM_CELLS), jnp.float32),
            jax.ShapeDtypeStruct((_NBLK, 1, BATCH), jnp.float32),
            jax.ShapeDtypeStruct((1, BATCH), jnp.float32),
        ],
        scratch_shapes=[
            pltpu.VMEM((8, BATCH), jnp.float32),
            pltpu.VMEM((BATCH, _BLK_CELLS), jnp.float32),
            pltpu.VMEM((_ROWS, 128), jnp.int32),
            pltpu.VMEM((_ROWS, 128), jnp.int32),
            pltpu.VMEM((_ROWS, 64), jnp.float32),
            pltpu.SemaphoreType.DMA,
        ],
    )(sdr_f, mod2, prev_f, vol2, cons2, dist3)

    f2_i32 = f2[:, 0, 0].astype(jnp.int32)

    pred_f = pl.pallas_call(
        _phase_b,
        grid=(_NBLK,),
        in_specs=[
            pl.BlockSpec(memory_space=pltpu.SMEM),
            pl.BlockSpec((BATCH, _BLK_COLS), lambda i: (0, i)),
            pl.BlockSpec((1, BATCH), lambda i: (0, 0)),
            pl.BlockSpec((BATCH, NUM_CELLS), lambda i: (0, 0)),
            pl.BlockSpec((BATCH, _BLK_CELLS), lambda i: (0, i)),
            pl.BlockSpec((BATCH, NUM_CELLS), lambda i: (0, 0)),
            pl.BlockSpec(memory_space=pl.ANY),
            pl.BlockSpec(memory_space=pl.ANY),
            pl.BlockSpec(memory_space=pl.ANY),
        ],
        out_specs=pl.BlockSpec((BATCH, _BLK_CELLS), lambda i: (0, i)),
        out_shape=jax.ShapeDtypeStruct((BATCH, NUM_CELLS), jnp.float32),
        scratch_shapes=[
            pltpu.VMEM((_ROWS, 128), jnp.int32),
            pltpu.VMEM((_ROWS, 128), jnp.float32),
            pltpu.VMEM((_ROWS, 128), jnp.float32),
            pltpu.VMEM((_ROWS, 128), jnp.float32),
            pltpu.VMEM((_ROWS, 64), jnp.float32),
            pltpu.SemaphoreType.DMA,
        ],
    )(f2_i32, sdr_f, mod2, new_active_f, new_active_f, prev_f,
      dist3, vol3, cons3)

    return (new_active_f.astype(bool), pred_f.astype(bool),
            acc.reshape(BATCH))


# speedup vs baseline: 1.0067x; 1.0067x over previous
import jax
import jax.numpy as jnp
from jax import lax
from jax.experimental import pallas as pl
from jax.experimental.pallas import tpu as pltpu

COLUMNS = 2048
CELLS_PER_COLUMN = 8
NUM_CELLS = COLUMNS * CELLS_PER_COLUMN
SEGMENTS = 16
SYNAPSES = 32
SEGSYN = SEGMENTS * SYNAPSES
ACTIVATION_THRESHOLD = 5
CONNECTED_PERMANENCE = 0.8
VOLATILE_LR = 0.1
BATCH = 16
_BLK_CELLS = 1024
_BLK_COLS = _BLK_CELLS // CELLS_PER_COLUMN
_NBLK = NUM_CELLS // _BLK_CELLS
_ROWS = _BLK_CELLS * SEGSYN // 128
_RCHUNK = 32


def _expand_mat():
    c = lax.broadcasted_iota(jnp.int32, (_BLK_COLS, _BLK_CELLS), 0)
    n = lax.broadcasted_iota(jnp.int32, (_BLK_COLS, _BLK_CELLS), 1) // CELLS_PER_COLUMN
    return (c == n).astype(jnp.bfloat16)


def _reduce_mat():
    n = lax.broadcasted_iota(jnp.int32, (_BLK_CELLS, _BLK_COLS), 0) // CELLS_PER_COLUMN
    c = lax.broadcasted_iota(jnp.int32, (_BLK_CELLS, _BLK_COLS), 1)
    return (n == c).astype(jnp.bfloat16)


def _seg_mat():
    j = lax.broadcasted_iota(jnp.int32, (SEGSYN, SEGMENTS), 0) // SYNAPSES
    s = lax.broadcasted_iota(jnp.int32, (SEGSYN, SEGMENTS), 1)
    return (j == s).astype(jnp.bfloat16)


def _pack_rows(act):
    p = jnp.zeros((1, NUM_CELLS), jnp.float32)
    for b in range(BATCH):
        p = p + act[b:b + 1, :] * float(1 << b)
    return p.reshape(128, 128)


def _byte_tables(packed):
    pi = packed.astype(jnp.int32)
    lo = (pi & 255).astype(jnp.bfloat16)
    hi = ((pi >> 8) & 255).astype(jnp.bfloat16)
    return lo, hi


def _onehots(idx):
    lane = lax.broadcasted_iota(jnp.int32, (_RCHUNK, 128), 1)
    oh_hi = ((idx >> 7) == lane).astype(jnp.bfloat16)
    oh_lo = ((idx & 127) == lane).astype(jnp.float32)
    return oh_hi, oh_lo


def _gather_tbl(oh_hi, oh_lo, tbl):
    rows = jnp.dot(oh_hi, tbl, preferred_element_type=jnp.float32)
    return jnp.sum(rows * oh_lo, axis=1, keepdims=True)


def _bhot(b):
    lane = lax.broadcasted_iota(jnp.int32, (1, BATCH), 1)
    return (lane == b).astype(jnp.float32)


def _seg_to_cells(ovacc):
    predseg = (ovacc >= ACTIVATION_THRESHOLD).astype(jnp.bfloat16)
    l64 = lax.broadcasted_iota(jnp.int32, (64, BATCH), 0)
    bix = lax.broadcasted_iota(jnp.int32, (64, BATCH), 1)
    msel = ((l64 & 15) == bix).astype(jnp.bfloat16)
    scnt = jnp.dot(predseg, msel, preferred_element_type=jnp.float32)
    cix = lax.broadcasted_iota(jnp.int32, (_BLK_CELLS, _ROWS), 0)
    rix = lax.broadcasted_iota(jnp.int32, (_BLK_CELLS, _ROWS), 1) // 4
    m4 = (cix == rix).astype(jnp.bfloat16)
    cellcnt = jnp.dot(m4, scnt.astype(jnp.bfloat16),
                      preferred_element_type=jnp.float32)
    padded = jnp.concatenate(
        [cellcnt, jnp.zeros((_BLK_CELLS, 128 - BATCH), jnp.float32)], axis=1)
    t = jnp.transpose(padded)
    return (t[0:BATCH, :] > 0).astype(jnp.float32)


def _accumulate_overlaps(dist_s, mask_fn, tlo, thi, ovacc_ref):
    def body(t, _):
        r0 = t * _RCHUNK
        dchunk = dist_s[pl.ds(r0, _RCHUNK), :]
        accs = []
        for sg in range(4):
            acc16 = jnp.zeros((_RCHUNK, BATCH), jnp.float32)
            for j in range(sg * 32, (sg + 1) * 32):
                idx = dchunk[:, j:j + 1]
                oh_hi, oh_lo = _onehots(idx)
                glo = _gather_tbl(oh_hi, oh_lo, tlo)
                ghi = _gather_tbl(oh_hi, oh_lo, thi)
                g = (glo + 256.0 * ghi).astype(jnp.int32)
                m = g * mask_fn(r0, j, oh_hi, oh_lo)
                for b in range(BATCH):
                    acc16 = acc16 + ((m >> b) & 1).astype(jnp.float32) * _bhot(b)
            accs.append(acc16)
        ovacc_ref[pl.ds(r0, _RCHUNK), :] = jnp.concatenate(accs, axis=1)
        return 0

    lax.fori_loop(0, _ROWS // _RCHUNK, body, 0)


def _column_phase(pred_now, sdr, na_out_ref):
    mexp = _expand_mat()
    mred = _reduce_mat()
    colcnt = jnp.dot(pred_now.astype(jnp.bfloat16), mred,
                     preferred_element_type=jnp.float32)
    col_has = colcnt > 0
    colpred_exp = jnp.dot(col_has.astype(jnp.bfloat16), mexp,
                          preferred_element_type=jnp.float32)
    sdr_exp = jnp.dot(sdr.astype(jnp.bfloat16), mexp,
                      preferred_element_type=jnp.float32)
    na = sdr_exp * jnp.where(colpred_exp > 0, pred_now, 1.0)
    na_out_ref[...] = na.astype(jnp.int8)
    return jnp.sum(sdr, axis=1), jnp.sum(jnp.where(col_has, sdr, 0.0), axis=1)


def _phase_a(sdr_ref, mod_ref, prev_ref, vol_ref, cons_ref, dist_ref,
             na_ref, f2_ref, acc_ref,
             accs_ref, pred_s, prev_s, dist_s, conn_s, ovacc_ref, sem):
    i = pl.program_id(0)
    delta = VOLATILE_LR * (jnp.sum(mod_ref[...]) / BATCH)
    vol = vol_ref[...]
    cons = cons_ref[...]
    eff = vol + cons
    kmat = _seg_mat()
    conn = eff >= CONNECTED_PERMANENCE
    cnt1 = jnp.dot(conn.astype(jnp.bfloat16), kmat,
                   preferred_element_type=jnp.float32)
    cand2 = (jnp.clip(vol + delta, 0.0, 1.0) + cons + 1e-5) >= CONNECTED_PERMANENCE
    cnt2 = jnp.dot(cand2.astype(jnp.bfloat16), kmat,
                   preferred_element_type=jnp.float32)
    n_flag1 = jnp.sum((cnt1 >= ACTIVATION_THRESHOLD).astype(jnp.float32))
    n_flag2 = jnp.sum((cnt2 >= ACTIVATION_THRESHOLD).astype(jnp.float32))
    f2_ref[...] = jnp.full((1, 1, BATCH), n_flag2, jnp.float32)

    pred_s[...] = jnp.zeros((BATCH, _BLK_CELLS), jnp.float32)

    @pl.when(n_flag1 > 0)
    def _():
        cp = pltpu.make_async_copy(
            dist_ref.at[pl.ds(i * _ROWS, _ROWS)], dist_s, sem)
        cp.start()
        cp.wait()
        cp2 = pltpu.make_async_copy(prev_ref, prev_s, sem)
        cp2.start()
        cp2.wait()
        tlo, thi = _byte_tables(_pack_rows(prev_s[...]))
        conn_s[...] = conn.astype(jnp.int32).reshape(_ROWS, 128)

        def mask_fn(r0, j, oh_hi, oh_lo):
            return conn_s[pl.ds(r0, _RCHUNK), j:j + 1]

        _accumulate_overlaps(dist_s, mask_fn, tlo, thi, ovacc_ref)
        pred_s[...] = _seg_to_cells(ovacc_ref[...])

    pred_now = pred_s[...]
    na_part, np_part = _column_phase(pred_now, sdr_ref[...], na_ref)

    @pl.when(i == 0)
    def _():
        accs_ref[...] = jnp.zeros((8, BATCH), jnp.float32)

    accs_ref[0:1, :] += na_part.reshape(1, BATCH)
    accs_ref[1:2, :] += np_part.reshape(1, BATCH)

    @pl.when(i == _NBLK - 1)
    def _():
        nact = accs_ref[0:1, :]
        npred = accs_ref[1:2, :]
        acc_ref[...] = jnp.where(nact > 0,
                                 npred / jnp.maximum(nact, 1.0), 1.0)


def _phase_b(f2_ref, sdr_ref, mod_ref, na_full_ref, prev_ref,
             dist_ref, vol_ref, cons_ref,
             pred_ref,
             na_s, prev_s, dist_s, vol_s, cons_s, w_s, ovacc_ref, sem):
    i = pl.program_id(0)
    pred_ref[...] = jnp.zeros((BATCH, _BLK_CELLS), jnp.int8)

    @pl.when(f2_ref[i] > 0)
    def _():
        base = i * _ROWS
        for src, dst in ((dist_ref.at[pl.ds(base, _ROWS)], dist_s),
                         (vol_ref.at[pl.ds(base, _ROWS)], vol_s),
                         (cons_ref.at[pl.ds(base, _ROWS)], cons_s),
                         (na_full_ref, na_s),
                         (prev_ref, prev_s)):
            cp = pltpu.make_async_copy(src, dst, sem)
            cp.start()
            cp.wait()

        delta = VOLATILE_LR * (jnp.sum(mod_ref[...]) / BATCH)
        na_full = na_s[...].astype(jnp.float32)
        tlo, thi = _byte_tables(_pack_rows(na_full))
        prevf = prev_s[...]
        pany = jnp.zeros((1, NUM_CELLS), jnp.float32)
        for b in range(BATCH):
            pany = pany + prevf[b:b + 1, :]
        tpa = (pany > 0).astype(jnp.bfloat16).reshape(128, 128)

        na_blk = na_s[:, pl.ds(i * _BLK_CELLS, _BLK_CELLS)].astype(jnp.float32)
        wcnt = lax.dot_general(
            na_blk.astype(jnp.bfloat16), jnp.ones((BATCH, 1), jnp.bfloat16),
            (((0,), (0,)), ((), ())), preferred_element_type=jnp.float32)
        wcol = (wcnt > 0).astype(jnp.bfloat16)
        rix = lax.broadcasted_iota(jnp.int32, (_ROWS, _BLK_CELLS), 0) // 4
        cix = lax.broadcasted_iota(jnp.int32, (_ROWS, _BLK_CELLS), 1)
        m4t = (rix == cix).astype(jnp.bfloat16)
        w4096 = jnp.dot(m4t, wcol, preferred_element_type=jnp.float32)
        w_s[...] = w4096 * jnp.ones((1, 128), jnp.float32)

        def mask_fn(r0, j, oh_hi, oh_lo):
            pp = _gather_tbl(oh_hi, oh_lo, tpa)
            vol_c = vol_s[pl.ds(r0, _RCHUNK), j:j + 1]
            cons_c = cons_s[pl.ds(r0, _RCHUNK), j:j + 1]
            w_c = w_s[pl.ds(r0, _RCHUNK), j:j + 1]
            nv = jnp.clip(
                vol_c + delta * w_c * pp - 0.1 * delta * w_c * (1.0 - pp),
                0.0, 1.0)
            return ((nv + cons_c) >= CONNECTED_PERMANENCE).astype(jnp.int32)

        _accumulate_overlaps(dist_s, mask_fn, tlo, thi, ovacc_ref)
        pred2 = _seg_to_cells(ovacc_ref[...])
        sdr_exp = jnp.dot(sdr_ref[...].astype(jnp.bfloat16), _expand_mat(),
                          preferred_element_type=jnp.float32)
        pred_ref[...] = (pred2 * sdr_exp).astype(jnp.int8)


def kernel(sdr_batch, modulation_signal_batch, prev_active_cells,
           distal_connections, volatile_permanences, consolidated_permanences):
    sdr_f = sdr_batch.astype(jnp.float32)
    mod2 = modulation_signal_batch.reshape(1, BATCH)
    prev_f = prev_active_cells.astype(jnp.float32)
    vol2 = volatile_permanences.reshape(NUM_CELLS, SEGSYN)
    cons2 = consolidated_permanences.reshape(NUM_CELLS, SEGSYN)
    dist_i32 = distal_connections.astype(jnp.int32)
    nrows = NUM_CELLS * SEGSYN // 128
    dist3 = dist_i32.reshape(nrows, 128)
    vol3 = volatile_permanences.reshape(nrows, 128)
    cons3 = consolidated_permanences.reshape(nrows, 128)

    new_active_i8, f2, acc = pl.pallas_call(
        _phase_a,
        grid=(_NBLK,),
        in_specs=[
            pl.BlockSpec((BATCH, _BLK_COLS), lambda i: (0, i)),
            pl.BlockSpec((1, BATCH), lambda i: (0, 0)),
            pl.BlockSpec(memory_space=pl.ANY),
            pl.BlockSpec((_BLK_CELLS, SEGSYN), lambda i: (i, 0)),
            pl.BlockSpec((_BLK_CELLS, SEGSYN), lambda i: (i, 0)),
            pl.BlockSpec(memory_space=pl.ANY),
        ],
        out_specs=[
            pl.BlockSpec((BATCH, _BLK_CELLS), lambda i: (0, i)),
            pl.BlockSpec((1, 1, BATCH), lambda i: (i, 0, 0)),
            pl.BlockSpec((1, BATCH), lambda i: (0, 0)),
        ],
        out_shape=[
            jax.ShapeDtypeStruct((BATCH, NUM_CELLS), jnp.int8),
            jax.ShapeDtypeStruct((_NBLK, 1, BATCH), jnp.float32),
            jax.ShapeDtypeStruct((1, BATCH), jnp.float32),
        ],
        scratch_shapes=[
            pltpu.VMEM((8, BATCH), jnp.float32),
            pltpu.VMEM((BATCH, _BLK_CELLS), jnp.float32),
            pltpu.VMEM((BATCH, NUM_CELLS), jnp.float32),
            pltpu.VMEM((_ROWS, 128), jnp.int32),
            pltpu.VMEM((_ROWS, 128), jnp.int32),
            pltpu.VMEM((_ROWS, 64), jnp.float32),
            pltpu.SemaphoreType.DMA,
        ],
    )(sdr_f, mod2, prev_f, vol2, cons2, dist3)

    f2_i32 = f2[:, 0, 0].astype(jnp.int32)

    pred_f = pl.pallas_call(
        _phase_b,
        grid=(_NBLK,),
        in_specs=[
            pl.BlockSpec(memory_space=pltpu.SMEM),
            pl.BlockSpec((BATCH, _BLK_COLS), lambda i: (0, i)),
            pl.BlockSpec((1, BATCH), lambda i: (0, 0)),
            pl.BlockSpec(memory_space=pl.ANY),
            pl.BlockSpec(memory_space=pl.ANY),
            pl.BlockSpec(memory_space=pl.ANY),
            pl.BlockSpec(memory_space=pl.ANY),
            pl.BlockSpec(memory_space=pl.ANY),
        ],
        out_specs=pl.BlockSpec((BATCH, _BLK_CELLS), lambda i: (0, i)),
        out_shape=jax.ShapeDtypeStruct((BATCH, NUM_CELLS), jnp.int8),
        scratch_shapes=[
            pltpu.VMEM((BATCH, NUM_CELLS), jnp.int8),
            pltpu.VMEM((BATCH, NUM_CELLS), jnp.float32),
            pltpu.VMEM((_ROWS, 128), jnp.int32),
            pltpu.VMEM((_ROWS, 128), jnp.float32),
            pltpu.VMEM((_ROWS, 128), jnp.float32),
            pltpu.VMEM((_ROWS, 128), jnp.float32),
            pltpu.VMEM((_ROWS, 64), jnp.float32),
            pltpu.SemaphoreType.DMA,
        ],
    )(f2_i32, sdr_f, mod2, new_active_i8, prev_f,
      dist3, vol3, cons3)

    return (new_active_i8.astype(bool), pred_f.astype(bool),
            acc.reshape(BATCH))
